# baseline (speedup 1.0000x reference)
import jax
import jax.numpy as jnp
from jax import lax
from jax.experimental import pallas as pl
from jax.experimental.pallas import tpu as pltpu
from jax.experimental.pallas import tpu_sc as plsc

_ROWS, _COLS = 16384, 1024
_NW = 32
_ROWS_PER_W = _ROWS // _NW
_CHUNK = 32
_ITERS = _ROWS_PER_W // _CHUNK
_NBUF = 3


def _sc_copy_body(x_hbm, out_hbm, b0, b1, b2, i0, i1, i2, o0, o1, o2):
    wid = lax.axis_index("s") * 2 + lax.axis_index("c")
    base = wid * _ROWS_PER_W
    bufs = (b0, b1, b2)
    isems = (i0, i1, i2)
    osems = (o0, o1, o2)

    def in_copy(i):
        b = i % _NBUF
        return pltpu.make_async_copy(
            x_hbm.at[pl.ds(base + i * _CHUNK, _CHUNK), :], bufs[b], isems[b]
        )

    def out_copy(i):
        b = i % _NBUF
        return pltpu.make_async_copy(
            bufs[b], out_hbm.at[pl.ds(base + i * _CHUNK, _CHUNK), :], osems[b]
        )

    ins = {0: in_copy(0), 1: in_copy(1)}
    ins[0].start()
    ins[1].start()
    outs = {}
    for i in range(_ITERS):
        ins[i].wait()
        outs[i] = out_copy(i)
        outs[i].start()
        nxt = i + 2
        if nxt < _ITERS:
            prev = nxt - _NBUF
            if prev >= 0:
                outs[prev].wait()
            ins[nxt] = in_copy(nxt)
            ins[nxt].start()
    outs[_ITERS - 2].wait()
    outs[_ITERS - 1].wait()


def kernel(x):
    mesh = plsc.VectorSubcoreMesh(core_axis_name="c", subcore_axis_name="s")
    fn = pl.kernel(
        _sc_copy_body,
        out_type=jax.ShapeDtypeStruct((_ROWS, _COLS), jnp.float32),
        mesh=mesh,
        scratch_types=[
            pltpu.VMEM((_CHUNK, _COLS), jnp.float32),
            pltpu.VMEM((_CHUNK, _COLS), jnp.float32),
            pltpu.VMEM((_CHUNK, _COLS), jnp.float32),
            pltpu.SemaphoreType.DMA,
            pltpu.SemaphoreType.DMA,
            pltpu.SemaphoreType.DMA,
            pltpu.SemaphoreType.DMA,
            pltpu.SemaphoreType.DMA,
            pltpu.SemaphoreType.DMA,
        ],
    )
    return fn(x)


# speedup vs baseline: 1.6029x; 1.6029x over previous
import jax
import jax.numpy as jnp
from jax.experimental import pallas as pl

_BLOCK_ROWS = 2048


def _copy_body(x_ref, o_ref):
    o_ref[...] = x_ref[...]


def kernel(x):
    rows, cols = x.shape
    grid = (rows // _BLOCK_ROWS,)
    return pl.pallas_call(
        _copy_body,
        grid=grid,
        in_specs=[pl.BlockSpec((_BLOCK_ROWS, cols), lambda i: (i, 0))],
        out_specs=pl.BlockSpec((_BLOCK_ROWS, cols), lambda i: (i, 0)),
        out_shape=jax.ShapeDtypeStruct((rows, cols), x.dtype),
    )(x)
